# baseline (speedup 1.0000x reference)
import functools

import jax
import jax.numpy as jnp
from jax import lax
from jax.experimental import pallas as pl
from jax.experimental.pallas import tpu as pltpu, tpu_sc as plsc

N = 10000
E = 320000
D = 128
NP = 10240
BR = 1280
NG = NP // BR
NC, NS, L = 2, 16, 16
NW = NC * NS
EC = E // NW
R = 80
NCH = EC // R
NEG = -1.0e30

f32 = jnp.float32
i32 = jnp.int32


def _mm_body(x_ref, w_ref, o_ref):
    o_ref[...] = jnp.dot(x_ref[...], w_ref[...], preferred_element_type=f32)


def _tc_matmul(xp, wa):
    return pl.pallas_call(
        _mm_body,
        grid=(NG,),
        in_specs=[
            pl.BlockSpec((BR, D), lambda i: (i, 0)),
            pl.BlockSpec((D, 2 * D), lambda i: (0, 0)),
        ],
        out_specs=pl.BlockSpec((BR, 2 * D), lambda i: (i, 0)),
        out_shape=jax.ShapeDtypeStruct((NP, 2 * D), f32),
    )(xp, wa)


def _cmax_body(p_ref, o_ref):
    o_ref[...] = jnp.max(p_ref[...], axis=0).reshape(1, 1, BR)


def _tc_combine_c(parts):
    out = pl.pallas_call(
        _cmax_body,
        grid=(NG,),
        in_specs=[pl.BlockSpec((NW, BR), lambda i: (0, i))],
        out_specs=pl.BlockSpec((1, 1, BR), lambda i: (i, 0, 0)),
        out_shape=jax.ShapeDtypeStruct((NG, 1, BR), f32),
    )(parts)
    return out.reshape(NP)


def _msgnorm(h, aggp, denp, s, b):
    den = denp[0, 0, :] + denp[0, 1, :]
    rden = 1.0 / (den + 1e-16)
    agg = jnp.concatenate(
        [aggp[0, 0] + aggp[1, 0], aggp[0, 1] + aggp[1, 1]], axis=1)
    agg = agg * rden[:, None]
    an = agg / (jnp.sqrt(jnp.sum(agg * agg, axis=1, keepdims=True)) + 1e-12)
    xn = jnp.sqrt(jnp.sum(h * h, axis=1, keepdims=True))
    return jnp.maximum(h + s * xn * an + b, 0.0)


def _comb_mm_body(ha_ref, ap_ref, dp_ref, w_ref, s_ref, b_ref, o_ref):
    h = ha_ref[:, :D]
    out = _msgnorm(h, ap_ref[...], dp_ref[...], s_ref[0, 0], b_ref[...])
    o_ref[...] = jnp.dot(out, w_ref[...], preferred_element_type=f32)


def _tc_combine_mm(ha, aggp, denp, wa, s, b):
    return pl.pallas_call(
        _comb_mm_body,
        grid=(NG,),
        in_specs=[
            pl.BlockSpec((BR, 2 * D), lambda i: (i, 0)),
            pl.BlockSpec((NC, 2, BR, DH), lambda i: (0, 0, i, 0)),
            pl.BlockSpec((1, NC, BR), lambda i: (i, 0, 0)),
            pl.BlockSpec((D, 2 * D), lambda i: (0, 0)),
            pl.BlockSpec((1, 1), lambda i: (0, 0)),
            pl.BlockSpec((1, D), lambda i: (0, 0)),
        ],
        out_specs=pl.BlockSpec((BR, 2 * D), lambda i: (i, 0)),
        out_shape=jax.ShapeDtypeStruct((NP, 2 * D), f32),
    )(ha, aggp, denp, wa, s.reshape(1, 1), b.reshape(1, D))


def _comb_ln_body(ha_ref, ap_ref, dp_ref, s_ref, b_ref, g_ref, lb_ref, o_ref):
    h = ha_ref[:, :D]
    out = _msgnorm(h, ap_ref[...], dp_ref[...], s_ref[0, 0], b_ref[...])
    mu = jnp.mean(out, axis=1, keepdims=True)
    var = jnp.mean((out - mu) ** 2, axis=1, keepdims=True)
    o_ref[...] = (out - mu) / jnp.sqrt(var + 1e-5) * g_ref[...] + lb_ref[...]


def _tc_combine_ln(ha, aggp, denp, s, b, ln_g, ln_b):
    return pl.pallas_call(
        _comb_ln_body,
        grid=(NG,),
        in_specs=[
            pl.BlockSpec((BR, 2 * D), lambda i: (i, 0)),
            pl.BlockSpec((NC, 2, BR, DH), lambda i: (0, 0, i, 0)),
            pl.BlockSpec((1, NC, BR), lambda i: (i, 0, 0)),
            pl.BlockSpec((1, 1), lambda i: (0, 0)),
            pl.BlockSpec((1, D), lambda i: (0, 0)),
            pl.BlockSpec((1, D), lambda i: (0, 0)),
            pl.BlockSpec((1, D), lambda i: (0, 0)),
        ],
        out_specs=pl.BlockSpec((BR, D), lambda i: (i, 0)),
        out_shape=jax.ShapeDtypeStruct((NP, D), f32),
    )(ha, aggp, denp, s.reshape(1, 1), b.reshape(1, D),
      ln_g.reshape(1, D), ln_b.reshape(1, D))


_MESH = plsc.VectorSubcoreMesh(
    core_axis_name="c", subcore_axis_name="s", num_cores=NC, num_subcores=NS)
_SC_PARAMS = pltpu.CompilerParams(
    needs_layout_passes=False, use_tc_tiling_on_sc=False)


def _fori(n, body):
    lax.fori_loop(0, n, lambda i, c: (body(i), c)[1], None)


def _sc_pass_a(als_hbm, ald_hbm, src_hbm, dst_hbm, e_hbm, cp_hbm,
               als_v, ald_v, src_v, dst_v, e_v, cp_v):
    cid = lax.axis_index("c")
    sid = lax.axis_index("s")
    wid = sid * NC + cid
    pltpu.sync_copy(als_hbm, als_v)
    pltpu.sync_copy(ald_hbm, ald_v)
    pltpu.sync_copy(src_hbm.at[wid], src_v)
    pltpu.sync_copy(dst_hbm.at[wid], dst_v)

    def init(i):
        cp_v[pl.ds(i * L, L)] = jnp.full((L,), NEG, f32)
    _fori(NP // L, init)

    def step(i):
        si = src_v[pl.ds(i * L, L)]
        di = dst_v[pl.ds(i * L, L)]
        ev = plsc.load_gather(als_v, [si]) + plsc.load_gather(ald_v, [di])
        ev = jnp.maximum(ev, 0.2 * ev)
        e_v[pl.ds(i * L, L)] = ev
        cur = plsc.load_gather(cp_v, [di])
        plsc.store_scatter(cp_v, [di], jnp.maximum(cur, ev))
    _fori(EC // L, step)

    pltpu.sync_copy(e_v, e_hbm.at[wid])
    pltpu.sync_copy(cp_v, cp_hbm.at[wid])


_sc_a = pl.kernel(
    _sc_pass_a,
    out_type=(jax.ShapeDtypeStruct((NW, EC), f32),
              jax.ShapeDtypeStruct((NW, NP), f32)),
    mesh=_MESH,
    compiler_params=_SC_PARAMS,
    scratch_types=[
        pltpu.VMEM((NP,), f32),
        pltpu.VMEM((NP,), f32),
        pltpu.VMEM((EC,), i32),
        pltpu.VMEM((EC,), i32),
        pltpu.VMEM((EC,), f32),
        pltpu.VMEM((NP,), f32),
    ],
)


DH = D // 2


def _sc_pass_bc(c_hbm, e_hbm, src_hbm, dst_hbm, h0_hbm, h1_hbm,
                dnp_hbm, agp_hbm,
                c_v, e2_v, ex2_v, src2_v, dst2_v, gbuf_v, sbuf_v, zd_v,
                agg_sh, den_sh, gsem0, gsem1, ssem0, ssem1, dsem):
    cid = lax.axis_index("c")
    sid = lax.axis_index("s")
    wid = sid * NC + cid
    slc = NP // NS

    pltpu.sync_copy(c_hbm, c_v)
    pltpu.sync_copy(e_hbm.at[wid], e2_v)
    pltpu.sync_copy(src_hbm.at[wid], src2_v)
    pltpu.sync_copy(dst_hbm.at[wid], dst2_v)

    def zden(i):
        zd_v[pl.ds(i * L, L)] = jnp.zeros((L,), f32)
    _fori(slc // L, zden)
    pltpu.sync_copy(zd_v, den_sh.at[pl.ds(sid * slc, slc)])
    plsc.subcore_barrier()

    pltpu.async_copy(h0_hbm.at[src2_v.at[0]], gbuf_v.at[0], gsem0)
    pltpu.async_copy(h0_hbm.at[src2_v.at[1]], gbuf_v.at[1], gsem1)

    def s1(i):
        for k in range(R // L):
            di = dst2_v[i, pl.ds(k * L, L)]
            cv = plsc.load_gather(c_v, [di])
            ev = e2_v[i, pl.ds(k * L, L)]
            ex2_v[i, pl.ds(k * L, L)] = jnp.exp(ev - cv)
        pltpu.async_copy(ex2_v.at[i], den_sh.at[dst2_v.at[i]], dsem, add=True)
    _fori(NCH, s1)

    for hf, h_hbm in ((0, h0_hbm), (1, h1_hbm)):
        def zrow(i):
            for k in range(DH // L):
                sbuf_v[1, i, pl.ds(k * L, L)] = jnp.zeros((L,), f32)
        _fori(R, zrow)
        for j in range(slc // R):
            pltpu.sync_copy(sbuf_v.at[1],
                            agg_sh.at[pl.ds(sid * slc + j * R, R)])
        plsc.subcore_barrier()

        gsem = (gsem0, gsem1)
        ssem = (ssem0, ssem1)
        if hf == 1:
            pltpu.async_copy(h_hbm.at[src2_v.at[0]], gbuf_v.at[0], gsem[0])
            pltpu.async_copy(h_hbm.at[src2_v.at[1]], gbuf_v.at[1], gsem[1])

        def chunk(j, b, wait_s, issue_g):
            pltpu.make_async_copy(
                h_hbm.at[src2_v.at[j]], gbuf_v.at[b], gsem[b]).wait()
            if wait_s:
                pltpu.make_async_copy(
                    sbuf_v.at[b], agg_sh.at[dst2_v.at[j - 2]],
                    ssem[b]).wait()

            def row(r):
                exb = plsc.load_gather(
                    ex2_v, [jnp.full((L,), j, i32), jnp.full((L,), r, i32)])
                for k in range(DH // L):
                    sbuf_v[b, r, pl.ds(k * L, L)] = (
                        gbuf_v[b, r, pl.ds(k * L, L)] * exb)
            _fori(R, row)
            pltpu.async_copy(sbuf_v.at[b], agg_sh.at[dst2_v.at[j]],
                             ssem[b], add=True)
            if issue_g:
                pltpu.async_copy(
                    h_hbm.at[src2_v.at[j + 2]], gbuf_v.at[b], gsem[b])

        chunk(0, 0, False, True)
        chunk(1, 1, False, True)

        def s2(g):
            chunk(2 * g, 0, True, True)
            chunk(2 * g + 1, 1, True, True)
        lax.fori_loop(1, (NCH - 3) // 2, lambda g, c: (s2(g), c)[1], None)
        chunk(NCH - 3, 0, True, True)
        chunk(NCH - 2, 1, True, False)
        chunk(NCH - 1, 0, True, False)
        pltpu.make_async_copy(
            sbuf_v.at[1], agg_sh.at[dst2_v.at[NCH - 2]], ssem[1]).wait()
        pltpu.make_async_copy(
            sbuf_v.at[0], agg_sh.at[dst2_v.at[NCH - 1]], ssem[0]).wait()

        plsc.subcore_barrier()
        pltpu.sync_copy(agg_sh.at[pl.ds(sid * slc, slc)],
                        agp_hbm.at[cid, hf, pl.ds(sid * slc, slc)])

    def drain(i):
        pltpu.make_async_copy(
            ex2_v.at[i], den_sh.at[dst2_v.at[i]], dsem).wait()
    _fori(NCH, drain)
    plsc.subcore_barrier()

    g = sid // 2
    off = (sid % 2) * slc
    pltpu.sync_copy(den_sh.at[pl.ds(sid * slc, slc)],
                    dnp_hbm.at[g, cid, pl.ds(off, slc)])


_sc_bc = pl.kernel(
    _sc_pass_bc,
    out_type=(jax.ShapeDtypeStruct((NG, NC, BR), f32),
              jax.ShapeDtypeStruct((NC, 2, NP, DH), f32)),
    mesh=_MESH,
    compiler_params=_SC_PARAMS,
    scratch_types=[
        pltpu.VMEM((NP,), f32),
        pltpu.VMEM((NCH, R), f32),
        pltpu.VMEM((NCH, R), f32),
        pltpu.VMEM((NCH, R), i32),
        pltpu.VMEM((NCH, R), i32),
        pltpu.VMEM((2, R, DH), f32),
        pltpu.VMEM((2, R, DH), f32),
        pltpu.VMEM((NP // NS,), f32),
        pltpu.VMEM_SHARED((NP, DH), f32),
        pltpu.VMEM_SHARED((NP,), f32),
        pltpu.SemaphoreType.DMA,
        pltpu.SemaphoreType.DMA,
        pltpu.SemaphoreType.DMA,
        pltpu.SemaphoreType.DMA,
        pltpu.SemaphoreType.DMA,
    ],
)


def _layer(xp, src_fl, dst_fl, src2, dst2, W, a_s, a_d):
    wa = jnp.concatenate(
        [W, (W @ a_s)[:, None], (W @ a_d)[:, None],
         jnp.zeros((D, D - 2), f32)], axis=1)
    ha = _tc_matmul(xp, wa)
    h = ha[:, :D]
    als = ha[:, D]
    ald = ha[:, D + 1]
    e, cpart = _sc_a(als, ald, src_fl, dst_fl)
    c = _tc_combine_c(cpart)
    e2 = e.reshape(NW, NCH, R)
    denp, aggp = _sc_bc(c, e2, src2, dst2, h[:, :DH], h[:, DH:])
    return ha, aggp, denp


@jax.jit
def kernel(x, edge_index, W1, a_s1, a_d1, s1, b1,
           W2, a_s2, a_d2, s2, b2, ln_g, ln_b):
    src = edge_index[0]
    dst = edge_index[1]
    src_fl = src.reshape(NW, EC)
    dst_fl = dst.reshape(NW, EC)
    src2 = src.reshape(NW, NCH, R)
    dst2 = dst.reshape(NW, NCH, R)
    xp = jnp.pad(x, ((0, NP - N), (0, 0)))

    ha1, aggp1, denp1 = _layer(xp, src_fl, dst_fl, src2, dst2, W1, a_s1, a_d1)
    wa2 = jnp.concatenate(
        [W2, (W2 @ a_s2)[:, None], (W2 @ a_d2)[:, None],
         jnp.zeros((D, D - 2), f32)], axis=1)
    ha2 = _tc_combine_mm(ha1, aggp1, denp1, wa2, s1, b1)
    h2 = ha2[:, :D]
    als2 = ha2[:, D]
    ald2 = ha2[:, D + 1]
    e, cpart = _sc_a(als2, ald2, src_fl, dst_fl)
    c = _tc_combine_c(cpart)
    denp2, aggp2 = _sc_bc(c, e.reshape(NW, NCH, R), src2, dst2,
                          h2[:, :DH], h2[:, DH:])
    y = _tc_combine_ln(ha2, aggp2, denp2, s2, b2, ln_g, ln_b)
    return y[:N]


# speedup vs baseline: 1.5033x; 1.5033x over previous
import functools

import jax
import jax.numpy as jnp
from jax import lax
from jax.experimental import pallas as pl
from jax.experimental.pallas import tpu as pltpu, tpu_sc as plsc

N = 10000
E = 320000
D = 128
NP = 10240
BR = 1280
NG = NP // BR
NC, NS, L = 2, 16, 16
NW = NC * NS
EC = E // NW
R = 80
NCH = EC // R
NEG = -1.0e30

f32 = jnp.float32
i32 = jnp.int32


def _mm_body(x_ref, w_ref, o_ref):
    o_ref[...] = jnp.dot(x_ref[...], w_ref[...], preferred_element_type=f32)


def _tc_matmul(xp, wa):
    return pl.pallas_call(
        _mm_body,
        grid=(NG,),
        in_specs=[
            pl.BlockSpec((BR, D), lambda i: (i, 0)),
            pl.BlockSpec((D, 2 * D), lambda i: (0, 0)),
        ],
        out_specs=pl.BlockSpec((BR, 2 * D), lambda i: (i, 0)),
        out_shape=jax.ShapeDtypeStruct((NP, 2 * D), f32),
    )(xp, wa)


def _cmax_body(p_ref, o_ref):
    o_ref[...] = jnp.max(p_ref[...], axis=0).reshape(1, 1, BR)


def _tc_combine_c(parts):
    out = pl.pallas_call(
        _cmax_body,
        grid=(NG,),
        in_specs=[pl.BlockSpec((NW, BR), lambda i: (0, i))],
        out_specs=pl.BlockSpec((1, 1, BR), lambda i: (i, 0, 0)),
        out_shape=jax.ShapeDtypeStruct((NG, 1, BR), f32),
    )(parts)
    return out.reshape(NP)


def _msgnorm(h, aggp, denp, s, b):
    den = denp[0, 0, :] + denp[0, 1, :]
    rden = 1.0 / (den + 1e-16)
    agg = jnp.concatenate(
        [aggp[0, 0] + aggp[1, 0], aggp[0, 1] + aggp[1, 1]], axis=1)
    agg = agg * rden[:, None]
    an = agg / (jnp.sqrt(jnp.sum(agg * agg, axis=1, keepdims=True)) + 1e-12)
    xn = jnp.sqrt(jnp.sum(h * h, axis=1, keepdims=True))
    return jnp.maximum(h + s * xn * an + b, 0.0)


def _comb_mm_body(ha_ref, ap_ref, dp_ref, w_ref, s_ref, b_ref, o_ref):
    h = ha_ref[:, :D]
    out = _msgnorm(h, ap_ref[...], dp_ref[...], s_ref[0, 0], b_ref[...])
    o_ref[...] = jnp.dot(out, w_ref[...], preferred_element_type=f32)


def _tc_combine_mm(ha, aggp, denp, wa, s, b):
    return pl.pallas_call(
        _comb_mm_body,
        grid=(NG,),
        in_specs=[
            pl.BlockSpec((BR, 2 * D), lambda i: (i, 0)),
            pl.BlockSpec((NC, 2, BR, DH), lambda i: (0, 0, i, 0)),
            pl.BlockSpec((1, NC, BR), lambda i: (i, 0, 0)),
            pl.BlockSpec((D, 2 * D), lambda i: (0, 0)),
            pl.BlockSpec((1, 1), lambda i: (0, 0)),
            pl.BlockSpec((1, D), lambda i: (0, 0)),
        ],
        out_specs=pl.BlockSpec((BR, 2 * D), lambda i: (i, 0)),
        out_shape=jax.ShapeDtypeStruct((NP, 2 * D), f32),
    )(ha, aggp, denp, wa, s.reshape(1, 1), b.reshape(1, D))


def _comb_ln_body(ha_ref, ap_ref, dp_ref, s_ref, b_ref, g_ref, lb_ref, o_ref):
    h = ha_ref[:, :D]
    out = _msgnorm(h, ap_ref[...], dp_ref[...], s_ref[0, 0], b_ref[...])
    mu = jnp.mean(out, axis=1, keepdims=True)
    var = jnp.mean((out - mu) ** 2, axis=1, keepdims=True)
    o_ref[...] = (out - mu) / jnp.sqrt(var + 1e-5) * g_ref[...] + lb_ref[...]


def _tc_combine_ln(ha, aggp, denp, s, b, ln_g, ln_b):
    return pl.pallas_call(
        _comb_ln_body,
        grid=(NG,),
        in_specs=[
            pl.BlockSpec((BR, 2 * D), lambda i: (i, 0)),
            pl.BlockSpec((NC, 2, BR, DH), lambda i: (0, 0, i, 0)),
            pl.BlockSpec((1, NC, BR), lambda i: (i, 0, 0)),
            pl.BlockSpec((1, 1), lambda i: (0, 0)),
            pl.BlockSpec((1, D), lambda i: (0, 0)),
            pl.BlockSpec((1, D), lambda i: (0, 0)),
            pl.BlockSpec((1, D), lambda i: (0, 0)),
        ],
        out_specs=pl.BlockSpec((BR, D), lambda i: (i, 0)),
        out_shape=jax.ShapeDtypeStruct((NP, D), f32),
    )(ha, aggp, denp, s.reshape(1, 1), b.reshape(1, D),
      ln_g.reshape(1, D), ln_b.reshape(1, D))


_MESH = plsc.VectorSubcoreMesh(
    core_axis_name="c", subcore_axis_name="s", num_cores=NC, num_subcores=NS)
_SC_PARAMS = pltpu.CompilerParams(
    needs_layout_passes=False, use_tc_tiling_on_sc=False)


def _fori(n, body):
    lax.fori_loop(0, n, lambda i, c: (body(i), c)[1], None)


def _sc_pass_a(als_hbm, ald_hbm, src_hbm, dst_hbm, e_hbm, cp_hbm,
               als_v, ald_v, src_v, dst_v, e_v, cp_v):
    cid = lax.axis_index("c")
    sid = lax.axis_index("s")
    wid = sid * NC + cid
    pltpu.sync_copy(als_hbm, als_v)
    pltpu.sync_copy(ald_hbm, ald_v)
    pltpu.sync_copy(src_hbm.at[wid], src_v)
    pltpu.sync_copy(dst_hbm.at[wid], dst_v)

    def init(i):
        cp_v[pl.ds(i * L, L)] = jnp.full((L,), NEG, f32)
    _fori(NP // L, init)

    def step(i):
        si = src_v[pl.ds(i * L, L)]
        di = dst_v[pl.ds(i * L, L)]
        ev = plsc.load_gather(als_v, [si]) + plsc.load_gather(ald_v, [di])
        ev = jnp.maximum(ev, 0.2 * ev)
        e_v[pl.ds(i * L, L)] = ev
        cur = plsc.load_gather(cp_v, [di])
        plsc.store_scatter(cp_v, [di], jnp.maximum(cur, ev))
    _fori(EC // L, step)

    pltpu.sync_copy(e_v, e_hbm.at[wid])
    pltpu.sync_copy(cp_v, cp_hbm.at[wid])


_sc_a = pl.kernel(
    _sc_pass_a,
    out_type=(jax.ShapeDtypeStruct((NW, EC), f32),
              jax.ShapeDtypeStruct((NW, NP), f32)),
    mesh=_MESH,
    compiler_params=_SC_PARAMS,
    scratch_types=[
        pltpu.VMEM((NP,), f32),
        pltpu.VMEM((NP,), f32),
        pltpu.VMEM((EC,), i32),
        pltpu.VMEM((EC,), i32),
        pltpu.VMEM((EC,), f32),
        pltpu.VMEM((NP,), f32),
    ],
)


DH = D // 2


def _sc_pass_bc(c_hbm, e_hbm, src_hbm, dst_hbm, h0_hbm, h1_hbm,
                dnp_hbm, agp_hbm,
                c_v, e2_v, ex2_v, src2_v, dst2_v, gbuf_v, sbuf_v, zd_v,
                agg_sh, den_sh, gsem0, gsem1, dsem):
    cid = lax.axis_index("c")
    sid = lax.axis_index("s")
    wid = sid * NC + cid
    slc = NP // NS

    pltpu.sync_copy(c_hbm, c_v)
    pltpu.sync_copy(e_hbm.at[wid], e2_v)
    pltpu.sync_copy(src_hbm.at[wid], src2_v)
    pltpu.sync_copy(dst_hbm.at[wid], dst2_v)

    def zden(i):
        zd_v[pl.ds(i * L, L)] = jnp.zeros((L,), f32)
    _fori(slc // L, zden)
    pltpu.sync_copy(zd_v, den_sh.at[pl.ds(sid * slc, slc)])
    plsc.subcore_barrier()

    pltpu.async_copy(h0_hbm.at[src2_v.at[0]], gbuf_v.at[0], gsem0)

    def s1(i):
        for k in range(R // L):
            di = dst2_v[i, pl.ds(k * L, L)]
            cv = plsc.load_gather(c_v, [di])
            ev = e2_v[i, pl.ds(k * L, L)]
            ex2_v[i, pl.ds(k * L, L)] = jnp.exp(ev - cv)
        pltpu.async_copy(ex2_v.at[i], den_sh.at[dst2_v.at[i]], dsem, add=True)
    _fori(NCH, s1)

    for hf, h_hbm in ((0, h0_hbm), (1, h1_hbm)):
        def zrow(i):
            for k in range(DH // L):
                sbuf_v[1, i, pl.ds(k * L, L)] = jnp.zeros((L,), f32)
        _fori(R, zrow)
        for j in range(slc // R):
            pltpu.sync_copy(sbuf_v.at[1],
                            agg_sh.at[pl.ds(sid * slc + j * R, R)])
        plsc.subcore_barrier()

        gsem = (gsem0, gsem1)
        if hf == 1:
            pltpu.async_copy(h_hbm.at[src2_v.at[0]], gbuf_v.at[0], gsem[0])

        def chunk(j, b, issue_next):
            pltpu.make_async_copy(
                h_hbm.at[src2_v.at[j]], gbuf_v.at[b], gsem[b]).wait()
            if issue_next:
                pltpu.async_copy(
                    h_hbm.at[src2_v.at[j + 1]], gbuf_v.at[1 - b],
                    gsem[1 - b])

            def row(r):
                exb = plsc.load_gather(
                    ex2_v, [jnp.full((L,), j, i32), jnp.full((L,), r, i32)])
                for k in range(DH // L):
                    gbuf_v[b, r, pl.ds(k * L, L)] = (
                        gbuf_v[b, r, pl.ds(k * L, L)] * exb)
            _fori(R, row)
            pltpu.sync_copy(gbuf_v.at[b], agg_sh.at[dst2_v.at[j]], add=True)

        def s2(g):
            chunk(2 * g, 0, True)
            chunk(2 * g + 1, 1, True)
        _fori((NCH - 1) // 2, s2)
        chunk(NCH - 1, 0, False)

        plsc.subcore_barrier()
        pltpu.sync_copy(agg_sh.at[pl.ds(sid * slc, slc)],
                        agp_hbm.at[cid, hf, pl.ds(sid * slc, slc)])

    def drain(i):
        pltpu.make_async_copy(
            ex2_v.at[i], den_sh.at[dst2_v.at[i]], dsem).wait()
    _fori(NCH, drain)
    plsc.subcore_barrier()

    g = sid // 2
    off = (sid % 2) * slc
    pltpu.sync_copy(den_sh.at[pl.ds(sid * slc, slc)],
                    dnp_hbm.at[g, cid, pl.ds(off, slc)])


_sc_bc = pl.kernel(
    _sc_pass_bc,
    out_type=(jax.ShapeDtypeStruct((NG, NC, BR), f32),
              jax.ShapeDtypeStruct((NC, 2, NP, DH), f32)),
    mesh=_MESH,
    compiler_params=_SC_PARAMS,
    scratch_types=[
        pltpu.VMEM((NP,), f32),
        pltpu.VMEM((NCH, R), f32),
        pltpu.VMEM((NCH, R), f32),
        pltpu.VMEM((NCH, R), i32),
        pltpu.VMEM((NCH, R), i32),
        pltpu.VMEM((2, R, DH), f32),
        pltpu.VMEM((2, R, DH), f32),
        pltpu.VMEM((NP // NS,), f32),
        pltpu.VMEM_SHARED((NP, DH), f32),
        pltpu.VMEM_SHARED((NP,), f32),
        pltpu.SemaphoreType.DMA,
        pltpu.SemaphoreType.DMA,
        pltpu.SemaphoreType.DMA,
    ],
)


def _layer(xp, src_fl, dst_fl, src2, dst2, W, a_s, a_d):
    wa = jnp.concatenate(
        [W, (W @ a_s)[:, None], (W @ a_d)[:, None],
         jnp.zeros((D, D - 2), f32)], axis=1)
    ha = _tc_matmul(xp, wa)
    h = ha[:, :D]
    als = ha[:, D]
    ald = ha[:, D + 1]
    e, cpart = _sc_a(als, ald, src_fl, dst_fl)
    c = _tc_combine_c(cpart)
    e2 = e.reshape(NW, NCH, R)
    denp, aggp = _sc_bc(c, e2, src2, dst2, h[:, :DH], h[:, DH:])
    return ha, aggp, denp


@jax.jit
def kernel(x, edge_index, W1, a_s1, a_d1, s1, b1,
           W2, a_s2, a_d2, s2, b2, ln_g, ln_b):
    src = edge_index[0]
    dst = edge_index[1]
    src_fl = src.reshape(NW, EC)
    dst_fl = dst.reshape(NW, EC)
    src2 = src.reshape(NW, NCH, R)
    dst2 = dst.reshape(NW, NCH, R)
    xp = jnp.pad(x, ((0, NP - N), (0, 0)))

    ha1, aggp1, denp1 = _layer(xp, src_fl, dst_fl, src2, dst2, W1, a_s1, a_d1)
    wa2 = jnp.concatenate(
        [W2, (W2 @ a_s2)[:, None], (W2 @ a_d2)[:, None],
         jnp.zeros((D, D - 2), f32)], axis=1)
    ha2 = _tc_combine_mm(ha1, aggp1, denp1, wa2, s1, b1)
    h2 = ha2[:, :D]
    als2 = ha2[:, D]
    ald2 = ha2[:, D + 1]
    e, cpart = _sc_a(als2, ald2, src_fl, dst_fl)
    c = _tc_combine_c(cpart)
    denp2, aggp2 = _sc_bc(c, e.reshape(NW, NCH, R), src2, dst2,
                          h2[:, :DH], h2[:, DH:])
    y = _tc_combine_ln(ha2, aggp2, denp2, s2, b2, ln_g, ln_b)
    return y[:N]
